# Initial kernel scaffold; baseline (speedup 1.0000x reference)
import jax
import jax.numpy as jnp
from jax.experimental import pallas as pl


def kernel(x, pos, edge_index, batch, node_W1, node_b1, node_W2, node_b2, conv_W, conv_Wih, conv_Whh, conv_bih, conv_bhh, lin1_W, lin1_b, lin2_W, lin2_b):
    raise NotImplementedError("write your pallas kernel here")


# speedup vs baseline: 2.4503x; 2.4503x over previous
import functools

import jax
import jax.numpy as jnp
import numpy as np
from jax import lax
from jax.experimental import pallas as pl
from jax.experimental.pallas import tpu as pltpu
from jax.experimental.pallas import tpu_sc as plsc

N = 10000
NPAD = 10240
E = 320000
H = 128
OUT = 64
G = 16
NC = 2
NS = 16
NW = NC * NS
K = 128
SHARD_SIZES = ([10080] * 11 + [9840] * 4 + [9760]) * 2
SHARD_OFFS = np.concatenate([[0], np.cumsum(SHARD_SIZES)])[:-1]
CAP = 10112
CHUNKS = CAP // K
ROWS_PER_SUB = NPAD // NS
RB = 256
GRID = NPAD // RB


def _dot(a, b):
    return jnp.dot(a, b, preferred_element_type=jnp.float32)


def _sc_agg_body(m_hbm, gsrc_hbm, gmeta_hbm, zeros_hbm, parts_hbm, side_hbm,
                 idx_v, meta_v, gbuf, fbuf, fidx, sidebuf, zbuf, sem):
    c = lax.axis_index("c")
    s = lax.axis_index("s")
    wid = s * NC + c
    base = s * ROWS_PER_SUB
    pbase = c * NPAD
    dump = pbase + N + s

    pltpu.sync_copy(zeros_hbm, zbuf)
    for r in range(4):
        pltpu.sync_copy(zbuf, parts_hbm.at[pl.ds(pbase + base + r * 160, 160)])
    pltpu.sync_copy(gsrc_hbm.at[wid], idx_v)
    pltpu.sync_copy(gmeta_hbm.at[wid], meta_v)
    plsc.subcore_barrier()

    iota16 = lax.broadcasted_iota(jnp.int32, (16,), 0)
    lane0 = iota16 == 0
    zvec = jnp.zeros((16,), jnp.float32)

    def reset_fidx():
        dv = jnp.full((16,), dump, jnp.int32)
        for q in range(8):
            fidx[0, pl.ds(16 * q, 16)] = dv

    reset_fidx()

    def edge_step(e, meta, carry):
        (a0, a1, a2, a3, a4, a5, a6, a7, prev, fcnt) = carry
        new_b = (meta & 1) == 1
        pf = (meta >> 1) & 1
        row = meta >> 2
        ev = jnp.full((16,), e, jnp.int32)
        v = [plsc.load_gather(gbuf, [ev, iota16 + 16 * q]) for q in range(8)]
        accs = (a0, a1, a2, a3, a4, a5, a6, a7)
        side_b = jnp.logical_and(new_b, pf == 1)
        int_b = jnp.logical_and(new_b, pf == 0)

        @pl.when(side_b)
        def _():
            for q in range(8):
                sidebuf[0, pl.ds(16 * q, 16)] = accs[q]

        def do_interior():
            tgt = jnp.where(prev < 0, dump, prev + pbase)
            fv = jnp.full((16,), fcnt, jnp.int32)
            for q in range(8):
                plsc.store_scatter(fbuf, [fv, iota16 + 16 * q], accs[q])
            plsc.store_scatter(
                fidx,
                [jnp.zeros((16,), jnp.int32),
                 jnp.full((16,), fcnt, jnp.int32)],
                jnp.full((16,), tgt, jnp.int32), mask=lane0)
            nf = fcnt + jnp.int32(1)

            def do_batch():
                pltpu.sync_copy(fbuf, parts_hbm.at[fidx.at[0]])
                reset_fidx()
                return jnp.int32(0)

            return lax.cond(nf == K, do_batch, lambda: nf)

        fcnt2 = lax.cond(int_b, do_interior, lambda: fcnt)
        accs2 = [jnp.where(new_b, v[q], accs[q] + v[q]) for q in range(8)]
        prev2 = jnp.where(new_b, row, prev)
        return tuple(accs2 + [prev2, fcnt2])

    def chunk_body(j, carry):
        pltpu.async_copy(m_hbm.at[idx_v.at[j]], gbuf, sem).wait()

        def grp_body(g, carry2):
            mv = meta_v[j, pl.ds(g * 16, 16)]
            out = carry2
            for lane in range(16):
                out = edge_step(g * 16 + lane, mv[lane], out)
            return out

        return lax.fori_loop(0, K // 16, grp_body, carry)

    init = tuple([zvec] * 8 + [jnp.int32(-1), jnp.int32(0)])
    fin = lax.fori_loop(0, CHUNKS, chunk_body, init)
    (a0, a1, a2, a3, a4, a5, a6, a7, prev, fcnt) = fin
    accs = (a0, a1, a2, a3, a4, a5, a6, a7)
    for q in range(8):
        sidebuf[1, pl.ds(16 * q, 16)] = accs[q]
    pltpu.sync_copy(sidebuf, side_hbm.at[wid])

    @pl.when(fcnt > 0)
    def _():
        pltpu.sync_copy(fbuf, parts_hbm.at[fidx.at[0]])


@functools.cache
def _get_sc_agg():
    return pl.kernel(
        _sc_agg_body,
        out_type=[jax.ShapeDtypeStruct((2 * NPAD, H), jnp.float32),
                  jax.ShapeDtypeStruct((NW, 2, H), jnp.float32)],
        mesh=plsc.VectorSubcoreMesh(core_axis_name="c", subcore_axis_name="s"),
        compiler_params=pltpu.CompilerParams(needs_layout_passes=False),
        scratch_types=[
            pltpu.VMEM((CHUNKS, K), jnp.int32),
            pltpu.VMEM((CHUNKS, K), jnp.int32),
            pltpu.VMEM((K, H), jnp.float32),
            pltpu.VMEM((K, H), jnp.float32),
            pltpu.VMEM((1, K), jnp.int32),
            pltpu.VMEM((2, H), jnp.float32),
            pltpu.VMEM((160, H), jnp.float32),
            pltpu.SemaphoreType.DMA,
        ],
    )


def _sc_merge_body(partsA_hbm, side_hbm, ids_hbm, partsB_hbm,
                   vbuf, mbuf, stage, ids_v, sem):
    c = lax.axis_index("c")
    s = lax.axis_index("s")
    off = c * NPAD + s * ROWS_PER_SUB
    for r in range(4):
        pltpu.sync_copy(partsA_hbm.at[pl.ds(off + r * 160, 160)], vbuf)
        pltpu.sync_copy(vbuf, partsB_hbm.at[pl.ds(off + r * 160, 160)])
    plsc.subcore_barrier()

    @pl.when(jnp.logical_and(c == 0, s == 0))
    def _():
        pltpu.sync_copy(side_hbm, mbuf)
        pltpu.sync_copy(ids_hbm, ids_v)

        idrows = [ids_v[r, pl.ds(0, 16)] for r in range(4)]
        zv = jnp.zeros((16,), jnp.float32)
        prev = jnp.int32(-1)
        accs = [zv] * 8
        for j in range(2 * NW):
            rid = idrows[j // 16][j % 16]
            v = [mbuf[j, pl.ds(16 * q, 16)] for q in range(8)]
            valid = rid >= 0
            newgrp = jnp.logical_and(valid, rid != prev)
            same = jnp.logical_and(valid, rid == prev)
            cur = list(accs)

            @pl.when(jnp.logical_and(newgrp, prev >= 0))
            def _(cur=cur, prev=prev):
                for q in range(8):
                    stage[0, pl.ds(16 * q, 16)] = cur[q]
                pltpu.sync_copy(stage, partsB_hbm.at[pl.ds(prev, 1)])

            accs = [jnp.where(newgrp, v[q],
                              jnp.where(same, cur[q] + v[q], cur[q]))
                    for q in range(8)]
            prev = jnp.where(newgrp, rid, prev)

        @pl.when(prev >= 0)
        def _():
            for q in range(8):
                stage[0, pl.ds(16 * q, 16)] = accs[q]
            pltpu.sync_copy(stage, partsB_hbm.at[pl.ds(prev, 1)])


@functools.cache
def _get_sc_merge():
    return pl.kernel(
        _sc_merge_body,
        out_type=jax.ShapeDtypeStruct((2 * NPAD, H), jnp.float32),
        mesh=plsc.VectorSubcoreMesh(core_axis_name="c", subcore_axis_name="s"),
        compiler_params=pltpu.CompilerParams(needs_layout_passes=False),
        scratch_types=[
            pltpu.VMEM((160, H), jnp.float32),
            pltpu.VMEM((2 * NW, H), jnp.float32),
            pltpu.VMEM((1, H), jnp.float32),
            pltpu.VMEM((4, 16), jnp.int32),
            pltpu.SemaphoreType.DMA,
        ],
    )


_BYPASS_MERGE = False


def _sc_agg(m, gsrc, gmeta, zeros, ids):
    parts, side = _get_sc_agg()(m, gsrc, gmeta, zeros)
    if _BYPASS_MERGE:
        return parts
    side64 = side.reshape(2 * NW, H)
    return _get_sc_merge()(parts, side64, ids)


def _row_mask(r, x):
    rows = r * RB + lax.broadcasted_iota(jnp.int32, (RB, 1), 0)
    return jnp.where(rows < N, x, 0.0)


def _pre_body(xc_ref, w1_ref, b1_ref, w2_ref, b2_ref, w0_ref, h_out, m_out):
    r = pl.program_id(0)
    t = jnp.maximum(_dot(xc_ref[...], w1_ref[...]) + b1_ref[...], 0.0)
    h = _dot(t, w2_ref[...]) + b2_ref[...]
    h = _row_mask(r, h)
    h_out[...] = h
    m_out[...] = _dot(h, w0_ref[...])


def _gru_core(r, p0_ref, p1_ref, h_ref, wihT_ref, whhT_ref, bih_ref, bhh_ref,
              relu):
    h = h_ref[...]
    agg = p0_ref[...] + p1_ref[...]
    gx = _dot(agg, wihT_ref[...]) + bih_ref[...]
    gh = _dot(h, whhT_ref[...]) + bhh_ref[...]
    rr = jax.nn.sigmoid(gx[:, :H] + gh[:, :H])
    zz = jax.nn.sigmoid(gx[:, H:2 * H] + gh[:, H:2 * H])
    nn = jnp.tanh(gx[:, 2 * H:] + rr * gh[:, 2 * H:])
    hnew = (1.0 - zz) * nn + zz * h
    if relu:
        hnew = jnp.maximum(hnew, 0.0)
    return _row_mask(r, hnew)


def _gru_body(p0_ref, p1_ref, h_ref, wihT_ref, whhT_ref, bih_ref, bhh_ref,
              wn_ref, h_out, m_out, *, relu):
    r = pl.program_id(0)
    hnew = _gru_core(r, p0_ref, p1_ref, h_ref, wihT_ref, whhT_ref,
                     bih_ref, bhh_ref, relu)
    h_out[...] = hnew
    m_out[...] = _dot(hnew, wn_ref[...])


def _tail_body(p0_ref, p1_ref, h_ref, wihT_ref, whhT_ref, bih_ref, bhh_ref,
               l1w_ref, l1b_ref, l2w_ref, l2b_ref, batch_ref, out_ref):
    r = pl.program_id(0)
    hnew = _gru_core(r, p0_ref, p1_ref, h_ref, wihT_ref, whhT_ref,
                     bih_ref, bhh_ref, True)
    y = jnp.maximum(_dot(hnew, l1w_ref[...]) + l1b_ref[...], 0.0)
    y = _dot(y, l2w_ref[...]) + l2b_ref[...]
    b = batch_ref[...].reshape(1, RB)
    oh = (lax.broadcasted_iota(jnp.int32, (G, RB), 0) == b).astype(jnp.float32)

    @pl.when(r == 0)
    def _():
        out_ref[...] = jnp.zeros_like(out_ref)

    out_ref[...] += _dot(oh, y)


def _full(shape):
    return pl.BlockSpec(shape, lambda r: (0,) * len(shape))


def _rowblk(w):
    return pl.BlockSpec((RB, w), lambda r: (r, 0))


_node_out = jax.ShapeDtypeStruct((NPAD, H), jnp.float32)

_pre = pl.pallas_call(
    _pre_body,
    grid=(GRID,),
    in_specs=[_rowblk(256), _full((256, H)), _full((1, H)), _full((H, H)),
              _full((1, H)), _full((H, H))],
    out_specs=[_rowblk(H), _rowblk(H)],
    out_shape=[_node_out, _node_out],
)


def _make_gru(relu):
    return pl.pallas_call(
        functools.partial(_gru_body, relu=relu),
        grid=(GRID,),
        in_specs=[_rowblk(H), _rowblk(H), _rowblk(H),
                  _full((H, 3 * H)), _full((H, 3 * H)),
                  _full((1, 3 * H)), _full((1, 3 * H)), _full((H, H))],
        out_specs=[_rowblk(H), _rowblk(H)],
        out_shape=[_node_out, _node_out],
    )


_gru_plain = _make_gru(False)
_gru_relu = _make_gru(True)

_tail = pl.pallas_call(
    _tail_body,
    grid=(GRID,),
    in_specs=[_rowblk(H), _rowblk(H), _rowblk(H),
              _full((H, 3 * H)), _full((H, 3 * H)),
              _full((1, 3 * H)), _full((1, 3 * H)),
              _full((H, OUT)), _full((1, OUT)),
              _full((OUT, OUT)), _full((1, OUT)),
              pl.BlockSpec((1, 1, RB), lambda r: (r, 0, 0))],
    out_specs=_full((G, OUT)),
    out_shape=jax.ShapeDtypeStruct((G, OUT), jnp.float32),
)


def _prep_edges(src, dst):
    order = jnp.argsort(dst, stable=True)
    ssrc = jnp.take(src, order)
    sdst = jnp.take(dst, order)
    offs = jnp.asarray(SHARD_OFFS, jnp.int32)
    sizes = jnp.asarray(SHARD_SIZES, jnp.int32)
    ar = jnp.arange(CAP, dtype=jnp.int32)
    pos = offs[:, None] + ar[None, :]
    valid = ar[None, :] < sizes[:, None]
    posc = jnp.clip(pos, 0, E - 1)
    gsrc = jnp.where(valid, ssrc[posc], N).astype(jnp.int32)
    d = sdst[posc]
    dprev = sdst[jnp.clip(pos - 1, 0, E - 1)]
    new = jnp.where(valid, (d != dprev) | (ar[None, :] == 0), False)
    row_ord = jnp.cumsum(new.astype(jnp.int32), axis=1)
    pfirst = new & (row_ord == 2)
    gmeta = jnp.where(valid,
                      (d.astype(jnp.int32) << 2)
                      | (pfirst.astype(jnp.int32) << 1)
                      | new.astype(jnp.int32),
                      0)
    gsrc = gsrc.reshape(NW, CHUNKS, K)
    gmeta = gmeta.reshape(NW, CHUNKS, K)
    first_id = sdst[offs]
    last_id = sdst[offs + sizes - 1]
    single = first_id == last_id
    first_id = jnp.where(single, -1, first_id)
    ids = jnp.stack([first_id, last_id], axis=1).reshape(4, 16).astype(jnp.int32)
    return gsrc, gmeta, ids


def kernel(x, pos, edge_index, batch, node_W1, node_b1, node_W2, node_b2,
           conv_W, conv_Wih, conv_Whh, conv_bih, conv_bhh,
           lin1_W, lin1_b, lin2_W, lin2_b):
    f32 = jnp.float32
    xc = jnp.concatenate([x, pos], axis=1)
    xc = jnp.pad(xc, ((0, NPAD - N), (0, 256 - (H + 3))))
    w1 = jnp.pad(node_W1, ((0, 256 - (H + 3)), (0, 0)))

    gsrc, gmeta, ids = _prep_edges(edge_index[0], edge_index[1])
    zeros_hbm = jnp.zeros((160, H), f32)

    batch3 = jnp.pad(batch, (0, NPAD - N), constant_values=-1)
    batch3 = batch3.reshape(GRID, 1, RB)

    wflat = conv_W.reshape(9, H, H)
    b1r = node_b1.reshape(1, H)
    b2r = node_b2.reshape(1, H)

    h, m = _pre(xc, w1, b1r, node_W2, b2r, wflat[0])
    out = None
    for step in range(9):
        c = step // 3
        parts = _sc_agg(m, gsrc, gmeta, zeros_hbm, ids)
        p0, p1 = parts[:NPAD], parts[NPAD:]
        wihT = conv_Wih[c].T
        whhT = conv_Whh[c].T
        bih = conv_bih[c].reshape(1, 3 * H)
        bhh = conv_bhh[c].reshape(1, 3 * H)
        if step < 8:
            gru = _gru_relu if step % 3 == 2 else _gru_plain
            h, m = gru(p0, p1, h, wihT, whhT, bih, bhh, wflat[step + 1])
        else:
            out = _tail(p0, p1, h, wihT, whhT, bih, bhh,
                        lin1_W, lin1_b.reshape(1, OUT),
                        lin2_W, lin2_b.reshape(1, OUT), batch3)
    return out


# speedup vs baseline: 2.4520x; 1.0007x over previous
import functools

import jax
import jax.numpy as jnp
import numpy as np
from jax import lax
from jax.experimental import pallas as pl
from jax.experimental.pallas import tpu as pltpu
from jax.experimental.pallas import tpu_sc as plsc

N = 10000
NPAD = 10240
E = 320000
H = 128
OUT = 64
G = 16
NC = 2
NS = 16
NW = NC * NS
K = 128
SHARD_SIZES = ([10080] * 11 + [9840] * 4 + [9760]) * 2
SHARD_OFFS = np.concatenate([[0], np.cumsum(SHARD_SIZES)])[:-1]
CAP = 10112
CHUNKS = CAP // K
ROWS_PER_SUB = NPAD // NS
RB = 256
GRID = NPAD // RB


def _dot(a, b):
    return jnp.dot(a, b, preferred_element_type=jnp.float32)


def _sc_agg_body(m_hbm, gsrc_hbm, gmeta_hbm, zeros_hbm, parts_hbm, side_hbm,
                 idx_v, meta_v, gbuf, fbuf, fidx, sidebuf, zbuf, sem):
    c = lax.axis_index("c")
    s = lax.axis_index("s")
    wid = s * NC + c
    base = s * ROWS_PER_SUB
    pbase = c * NPAD
    dump = pbase + N + s

    pltpu.sync_copy(zeros_hbm, zbuf)
    for r in range(4):
        pltpu.sync_copy(zbuf, parts_hbm.at[pl.ds(pbase + base + r * 160, 160)])
    pltpu.sync_copy(gsrc_hbm.at[wid], idx_v)
    pltpu.sync_copy(gmeta_hbm.at[wid], meta_v)
    plsc.subcore_barrier()

    iota16 = lax.broadcasted_iota(jnp.int32, (16,), 0)
    lane0 = iota16 == 0
    zvec = jnp.zeros((16,), jnp.float32)

    def reset_fidx():
        dv = jnp.full((16,), dump, jnp.int32)
        for q in range(8):
            fidx[0, pl.ds(16 * q, 16)] = dv

    reset_fidx()

    def edge_step(e, meta, carry):
        (a0, a1, a2, a3, a4, a5, a6, a7, prev, fcnt) = carry
        new_b = (meta & 1) == 1
        pf = (meta >> 1) & 1
        row = meta >> 2
        ev = jnp.full((16,), e, jnp.int32)
        v = [plsc.load_gather(gbuf, [ev, iota16 + 16 * q]) for q in range(8)]
        accs = (a0, a1, a2, a3, a4, a5, a6, a7)
        side_b = jnp.logical_and(new_b, pf == 1)
        int_b = jnp.logical_and(new_b, pf == 0)

        @pl.when(side_b)
        def _():
            for q in range(8):
                sidebuf[0, pl.ds(16 * q, 16)] = accs[q]

        def do_interior():
            tgt = jnp.where(prev < 0, dump, prev + pbase)
            fv = jnp.full((16,), fcnt, jnp.int32)
            for q in range(8):
                plsc.store_scatter(fbuf, [fv, iota16 + 16 * q], accs[q])
            plsc.store_scatter(
                fidx,
                [jnp.zeros((16,), jnp.int32),
                 jnp.full((16,), fcnt, jnp.int32)],
                jnp.full((16,), tgt, jnp.int32), mask=lane0)
            nf = fcnt + jnp.int32(1)

            def do_batch():
                pltpu.sync_copy(fbuf, parts_hbm.at[fidx.at[0]])
                reset_fidx()
                return jnp.int32(0)

            return lax.cond(nf == K, do_batch, lambda: nf)

        fcnt2 = lax.cond(int_b, do_interior, lambda: fcnt)
        accs2 = [jnp.where(new_b, v[q], accs[q] + v[q]) for q in range(8)]
        prev2 = jnp.where(new_b, row, prev)
        return tuple(accs2 + [prev2, fcnt2])

    def chunk_body(j, carry):
        pltpu.async_copy(m_hbm.at[idx_v.at[j]], gbuf, sem).wait()

        def grp_body(g, carry2):
            mv = meta_v[j, pl.ds(g * 16, 16)]
            out = carry2
            for lane in range(16):
                out = edge_step(g * 16 + lane, mv[lane], out)
            return out

        return lax.fori_loop(0, K // 16, grp_body, carry)

    init = tuple([zvec] * 8 + [jnp.int32(-1), jnp.int32(0)])
    fin = lax.fori_loop(0, CHUNKS, chunk_body, init)
    (a0, a1, a2, a3, a4, a5, a6, a7, prev, fcnt) = fin
    accs = (a0, a1, a2, a3, a4, a5, a6, a7)
    for q in range(8):
        sidebuf[1, pl.ds(16 * q, 16)] = accs[q]
    pltpu.sync_copy(sidebuf, side_hbm.at[wid])

    @pl.when(fcnt > 0)
    def _():
        pltpu.sync_copy(fbuf, parts_hbm.at[fidx.at[0]])


@functools.cache
def _get_sc_agg():
    return pl.kernel(
        _sc_agg_body,
        out_type=[jax.ShapeDtypeStruct((2 * NPAD, H), jnp.float32),
                  jax.ShapeDtypeStruct((NW, 2, H), jnp.float32)],
        mesh=plsc.VectorSubcoreMesh(core_axis_name="c", subcore_axis_name="s"),
        compiler_params=pltpu.CompilerParams(needs_layout_passes=False),
        scratch_types=[
            pltpu.VMEM((CHUNKS, K), jnp.int32),
            pltpu.VMEM((CHUNKS, K), jnp.int32),
            pltpu.VMEM((K, H), jnp.float32),
            pltpu.VMEM((K, H), jnp.float32),
            pltpu.VMEM((1, K), jnp.int32),
            pltpu.VMEM((2, H), jnp.float32),
            pltpu.VMEM((160, H), jnp.float32),
            pltpu.SemaphoreType.DMA,
        ],
    )


def _sc_merge_body(partsA_hbm, side_hbm, ids_hbm, partsB_hbm,
                   vbuf, mbuf, stage, ids_v, sem):
    c = lax.axis_index("c")
    s = lax.axis_index("s")
    off = c * NPAD + s * ROWS_PER_SUB
    for r in range(4):
        pltpu.sync_copy(partsA_hbm.at[pl.ds(off + r * 160, 160)], vbuf)
        pltpu.sync_copy(vbuf, partsB_hbm.at[pl.ds(off + r * 160, 160)])
    plsc.subcore_barrier()

    @pl.when(jnp.logical_and(c == 0, s == 0))
    def _():
        pltpu.sync_copy(side_hbm, mbuf)
        pltpu.sync_copy(ids_hbm, ids_v)

        idrows = [ids_v[r, pl.ds(0, 16)] for r in range(4)]
        zv = jnp.zeros((16,), jnp.float32)
        prev = jnp.int32(-1)
        accs = [zv] * 8
        for j in range(2 * NW):
            rid = idrows[j // 16][j % 16]
            v = [mbuf[j, pl.ds(16 * q, 16)] for q in range(8)]
            valid = rid >= 0
            newgrp = jnp.logical_and(valid, rid != prev)
            same = jnp.logical_and(valid, rid == prev)
            cur = list(accs)

            @pl.when(jnp.logical_and(newgrp, prev >= 0))
            def _(cur=cur, prev=prev):
                for q in range(8):
                    stage[0, pl.ds(16 * q, 16)] = cur[q]
                pltpu.sync_copy(stage, partsB_hbm.at[pl.ds(prev, 1)])

            accs = [jnp.where(newgrp, v[q],
                              jnp.where(same, cur[q] + v[q], cur[q]))
                    for q in range(8)]
            prev = jnp.where(newgrp, rid, prev)

        @pl.when(prev >= 0)
        def _():
            for q in range(8):
                stage[0, pl.ds(16 * q, 16)] = accs[q]
            pltpu.sync_copy(stage, partsB_hbm.at[pl.ds(prev, 1)])


@functools.cache
def _get_sc_merge():
    return pl.kernel(
        _sc_merge_body,
        out_type=jax.ShapeDtypeStruct((2 * NPAD, H), jnp.float32),
        mesh=plsc.VectorSubcoreMesh(core_axis_name="c", subcore_axis_name="s"),
        compiler_params=pltpu.CompilerParams(needs_layout_passes=False),
        scratch_types=[
            pltpu.VMEM((160, H), jnp.float32),
            pltpu.VMEM((2 * NW, H), jnp.float32),
            pltpu.VMEM((1, H), jnp.float32),
            pltpu.VMEM((4, 16), jnp.int32),
            pltpu.SemaphoreType.DMA,
        ],
    )


def _sc_agg(m, gsrc, gmeta, zeros, ids):
    parts, side = _get_sc_agg()(m, gsrc, gmeta, zeros)
    side64 = side.reshape(2 * NW, H)
    return _get_sc_merge()(parts, side64, ids)


def _row_mask(r, x):
    rows = r * RB + lax.broadcasted_iota(jnp.int32, (RB, 1), 0)
    return jnp.where(rows < N, x, 0.0)


def _pre_body(xc_ref, w1_ref, b1_ref, w2_ref, b2_ref, w0_ref, h_out, m_out):
    r = pl.program_id(0)
    t = jnp.maximum(_dot(xc_ref[...], w1_ref[...]) + b1_ref[...], 0.0)
    h = _dot(t, w2_ref[...]) + b2_ref[...]
    h = _row_mask(r, h)
    h_out[...] = h
    m_out[...] = _dot(h, w0_ref[...])


def _gru_core(r, p0_ref, p1_ref, h_ref, wihT_ref, whhT_ref, bih_ref, bhh_ref,
              relu):
    h = h_ref[...]
    agg = p0_ref[...] + p1_ref[...]
    gx = _dot(agg, wihT_ref[...]) + bih_ref[...]
    gh = _dot(h, whhT_ref[...]) + bhh_ref[...]
    rr = jax.nn.sigmoid(gx[:, :H] + gh[:, :H])
    zz = jax.nn.sigmoid(gx[:, H:2 * H] + gh[:, H:2 * H])
    nn = jnp.tanh(gx[:, 2 * H:] + rr * gh[:, 2 * H:])
    hnew = (1.0 - zz) * nn + zz * h
    if relu:
        hnew = jnp.maximum(hnew, 0.0)
    return _row_mask(r, hnew)


def _gru_body(p0_ref, p1_ref, h_ref, wihT_ref, whhT_ref, bih_ref, bhh_ref,
              wn_ref, h_out, m_out, *, relu):
    r = pl.program_id(0)
    hnew = _gru_core(r, p0_ref, p1_ref, h_ref, wihT_ref, whhT_ref,
                     bih_ref, bhh_ref, relu)
    h_out[...] = hnew
    m_out[...] = _dot(hnew, wn_ref[...])


def _tail_body(p0_ref, p1_ref, h_ref, wihT_ref, whhT_ref, bih_ref, bhh_ref,
               l1w_ref, l1b_ref, l2w_ref, l2b_ref, batch_ref, out_ref):
    r = pl.program_id(0)
    hnew = _gru_core(r, p0_ref, p1_ref, h_ref, wihT_ref, whhT_ref,
                     bih_ref, bhh_ref, True)
    y = jnp.maximum(_dot(hnew, l1w_ref[...]) + l1b_ref[...], 0.0)
    y = _dot(y, l2w_ref[...]) + l2b_ref[...]
    b = batch_ref[...].reshape(1, RB)
    oh = (lax.broadcasted_iota(jnp.int32, (G, RB), 0) == b).astype(jnp.float32)

    @pl.when(r == 0)
    def _():
        out_ref[...] = jnp.zeros_like(out_ref)

    out_ref[...] += _dot(oh, y)


def _full(shape):
    return pl.BlockSpec(shape, lambda r: (0,) * len(shape))


def _rowblk(w):
    return pl.BlockSpec((RB, w), lambda r: (r, 0))


_node_out = jax.ShapeDtypeStruct((NPAD, H), jnp.float32)

_pre = pl.pallas_call(
    _pre_body,
    grid=(GRID,),
    in_specs=[_rowblk(256), _full((256, H)), _full((1, H)), _full((H, H)),
              _full((1, H)), _full((H, H))],
    out_specs=[_rowblk(H), _rowblk(H)],
    out_shape=[_node_out, _node_out],
)


def _make_gru(relu):
    return pl.pallas_call(
        functools.partial(_gru_body, relu=relu),
        grid=(GRID,),
        in_specs=[_rowblk(H), _rowblk(H), _rowblk(H),
                  _full((H, 3 * H)), _full((H, 3 * H)),
                  _full((1, 3 * H)), _full((1, 3 * H)), _full((H, H))],
        out_specs=[_rowblk(H), _rowblk(H)],
        out_shape=[_node_out, _node_out],
    )


_gru_plain = _make_gru(False)
_gru_relu = _make_gru(True)

_tail = pl.pallas_call(
    _tail_body,
    grid=(GRID,),
    in_specs=[_rowblk(H), _rowblk(H), _rowblk(H),
              _full((H, 3 * H)), _full((H, 3 * H)),
              _full((1, 3 * H)), _full((1, 3 * H)),
              _full((H, OUT)), _full((1, OUT)),
              _full((OUT, OUT)), _full((1, OUT)),
              pl.BlockSpec((1, 1, RB), lambda r: (r, 0, 0))],
    out_specs=_full((G, OUT)),
    out_shape=jax.ShapeDtypeStruct((G, OUT), jnp.float32),
)


def _prep_edges(src, dst):
    order = jnp.argsort(dst, stable=True)
    ssrc = jnp.take(src, order)
    sdst = jnp.take(dst, order)
    offs = jnp.asarray(SHARD_OFFS, jnp.int32)
    sizes = jnp.asarray(SHARD_SIZES, jnp.int32)
    ar = jnp.arange(CAP, dtype=jnp.int32)
    pos = offs[:, None] + ar[None, :]
    valid = ar[None, :] < sizes[:, None]
    posc = jnp.clip(pos, 0, E - 1)
    gsrc = jnp.where(valid, ssrc[posc], N).astype(jnp.int32)
    d = sdst[posc]
    dprev = sdst[jnp.clip(pos - 1, 0, E - 1)]
    new = jnp.where(valid, (d != dprev) | (ar[None, :] == 0), False)
    row_ord = jnp.cumsum(new.astype(jnp.int32), axis=1)
    pfirst = new & (row_ord == 2)
    gmeta = jnp.where(valid,
                      (d.astype(jnp.int32) << 2)
                      | (pfirst.astype(jnp.int32) << 1)
                      | new.astype(jnp.int32),
                      0)
    gsrc = gsrc.reshape(NW, CHUNKS, K)
    gmeta = gmeta.reshape(NW, CHUNKS, K)
    first_id = sdst[offs]
    last_id = sdst[offs + sizes - 1]
    single = first_id == last_id
    first_id = jnp.where(single, -1, first_id)
    ids = jnp.stack([first_id, last_id], axis=1).reshape(4, 16).astype(jnp.int32)
    return gsrc, gmeta, ids


def kernel(x, pos, edge_index, batch, node_W1, node_b1, node_W2, node_b2,
           conv_W, conv_Wih, conv_Whh, conv_bih, conv_bhh,
           lin1_W, lin1_b, lin2_W, lin2_b):
    f32 = jnp.float32
    xc = jnp.concatenate([x, pos], axis=1)
    xc = jnp.pad(xc, ((0, NPAD - N), (0, 256 - (H + 3))))
    w1 = jnp.pad(node_W1, ((0, 256 - (H + 3)), (0, 0)))

    gsrc, gmeta, ids = _prep_edges(edge_index[0], edge_index[1])
    zeros_hbm = jnp.zeros((160, H), f32)

    batch3 = jnp.pad(batch, (0, NPAD - N), constant_values=-1)
    batch3 = batch3.reshape(GRID, 1, RB)

    wflat = conv_W.reshape(9, H, H)
    b1r = node_b1.reshape(1, H)
    b2r = node_b2.reshape(1, H)

    h, m = _pre(xc, w1, b1r, node_W2, b2r, wflat[0])
    out = None
    for step in range(9):
        c = step // 3
        parts = _sc_agg(m, gsrc, gmeta, zeros_hbm, ids)
        p0, p1 = parts[:NPAD], parts[NPAD:]
        wihT = conv_Wih[c].T
        whhT = conv_Whh[c].T
        bih = conv_bih[c].reshape(1, 3 * H)
        bhh = conv_bhh[c].reshape(1, 3 * H)
        if step < 8:
            gru = _gru_relu if step % 3 == 2 else _gru_plain
            h, m = gru(p0, p1, h, wihT, whhT, bih, bhh, wflat[step + 1])
        else:
            out = _tail(p0, p1, h, wihT, whhT, bih, bhh,
                        lin1_W, lin1_b.reshape(1, OUT),
                        lin2_W, lin2_b.reshape(1, OUT), batch3)
    return out
